# Initial kernel scaffold; baseline (speedup 1.0000x reference)
import jax
import jax.numpy as jnp
from jax.experimental import pallas as pl


def kernel(x, edge_index, batch, W1, a1_src, a1_dst, b1, W2, a2_src, a2_dst, b2, W3, a3_src, a3_dst, b3, Wfc, bfc):
    raise NotImplementedError("write your pallas kernel here")


# speedup vs baseline: 16.3478x; 16.3478x over previous
import functools

import jax
import jax.numpy as jnp
from jax import lax
from jax.experimental import pallas as pl
from jax.experimental.pallas import tpu as pltpu, tpu_sc as plsc

N = 10000
NP = 10240
G = 64
DH = 256
HALF = 128
E = 320000
NSUB = 16
CK = 128
CH = 162
EP = NSUB * CH * CK
DENR = NP // 16
R = 1024
NB = NP // R


def _tc_first_body(x_ref, w_ref, a_ref, h_ref, al_ref):
    h = jnp.dot(x_ref[...], w_ref[...], preferred_element_type=jnp.float32)
    h_ref[0] = h[:, :HALF]
    h_ref[1] = h[:, HALF:]
    al_ref[...] = jnp.dot(h, a_ref[...], preferred_element_type=jnp.float32)


def _tc_mid_body(x0_ref, x1_ref, b_ref, w_ref, a_ref, h_ref, al_ref):
    x = jnp.concatenate([x0_ref[...], x1_ref[...]], axis=1)
    x = jnp.maximum(x + b_ref[...], 0.0)
    h = jnp.dot(x, w_ref[...], preferred_element_type=jnp.float32)
    h_ref[0] = h[:, :HALF]
    h_ref[1] = h[:, HALF:]
    al_ref[...] = jnp.dot(h, a_ref[...], preferred_element_type=jnp.float32)


def _tc_layer_first(x_pad, W, A):
    return pl.pallas_call(
        _tc_first_body,
        grid=(NB,),
        in_specs=[
            pl.BlockSpec((R, W.shape[0]), lambda i: (i, 0)),
            pl.BlockSpec(W.shape, lambda i: (0, 0)),
            pl.BlockSpec(A.shape, lambda i: (0, 0)),
        ],
        out_specs=[
            pl.BlockSpec((2, R, HALF), lambda i: (0, i, 0)),
            pl.BlockSpec((R, 2), lambda i: (i, 0)),
        ],
        out_shape=[
            jax.ShapeDtypeStruct((2, NP, HALF), jnp.float32),
            jax.ShapeDtypeStruct((NP, 2), jnp.float32),
        ],
    )(x_pad, W, A)


def _tc_layer_mid(agg_flat, b_prev, W, A):
    return pl.pallas_call(
        _tc_mid_body,
        grid=(NB,),
        in_specs=[
            pl.BlockSpec((R, HALF), lambda i: (i, 0)),
            pl.BlockSpec((R, HALF), lambda i: (i + NB, 0)),
            pl.BlockSpec((1, DH), lambda i: (0, 0)),
            pl.BlockSpec(W.shape, lambda i: (0, 0)),
            pl.BlockSpec(A.shape, lambda i: (0, 0)),
        ],
        out_specs=[
            pl.BlockSpec((2, R, HALF), lambda i: (0, i, 0)),
            pl.BlockSpec((R, 2), lambda i: (i, 0)),
        ],
        out_shape=[
            jax.ShapeDtypeStruct((2, NP, HALF), jnp.float32),
            jax.ShapeDtypeStruct((NP, 2), jnp.float32),
        ],
    )(agg_flat, agg_flat, b_prev, W, A)


def _tc_final_body(x0_ref, x1_ref, b_ref, wfc_ref, bfc_ref, bt_ref,
                   out_ref, acc_ref):
    i = pl.program_id(0)
    x = jnp.concatenate([x0_ref[...], x1_ref[...]], axis=1)
    h = jnp.maximum(x + b_ref[...], 0.0)
    p = jnp.dot(h, wfc_ref[...], preferred_element_type=jnp.float32)
    bt = bt_ref[0, 0, :]
    gi = lax.broadcasted_iota(jnp.int32, (G, R), 0)
    onehot = (bt[None, :] == gi).astype(jnp.float32)
    contrib = jnp.dot(onehot, p, preferred_element_type=jnp.float32)

    @pl.when(i == 0)
    def _():
        acc_ref[...] = contrib

    @pl.when(i > 0)
    def _():
        acc_ref[...] = acc_ref[...] + contrib

    @pl.when(i == NB - 1)
    def _():
        out_ref[...] = acc_ref[...] + bfc_ref[...]


def _tc_final(agg_flat, b3, Wfc, bfc, batch3):
    return pl.pallas_call(
        _tc_final_body,
        grid=(NB,),
        in_specs=[
            pl.BlockSpec((R, HALF), lambda i: (i, 0)),
            pl.BlockSpec((R, HALF), lambda i: (i + NB, 0)),
            pl.BlockSpec((1, DH), lambda i: (0, 0)),
            pl.BlockSpec(Wfc.shape, lambda i: (0, 0)),
            pl.BlockSpec((1, Wfc.shape[1]), lambda i: (0, 0)),
            pl.BlockSpec((1, 1, R), lambda i: (i, 0, 0)),
        ],
        out_specs=pl.BlockSpec((G, Wfc.shape[1]), lambda i: (0, 0)),
        out_shape=jax.ShapeDtypeStruct((G, Wfc.shape[1]), jnp.float32),
        scratch_shapes=[pltpu.VMEM((G, Wfc.shape[1]), jnp.float32)],
    )(agg_flat, agg_flat, b3, Wfc, bfc, batch3)


def _sc_body(h_hbm, asrc_hbm, adst_hbm, src_hbm, dst_hbm, out_hbm,
             asrc_v, adst_v, srcb, dstb, exb, rows_v, dens_v, sem,
             acc_s, den_s):
    c = lax.axis_index("c")
    s = lax.axis_index("s")
    zf16 = jnp.zeros((16,), jnp.float32)
    off = c * NP

    pltpu.sync_copy(asrc_hbm, asrc_v)
    pltpu.sync_copy(adst_hbm, adst_v)

    @pl.loop(0, CK)
    def _(r):
        for f in range(8):
            rows_v[r, pl.ds(f * 16, 16)] = zf16

    @pl.loop(0, 640 // 16)
    def _(j):
        dens_v[pl.ds(j * 16, 16)] = zf16

    @pl.loop(0, 640 // CK)
    def _(k):
        pltpu.sync_copy(rows_v, acc_s.at[pl.ds(s * 640 + k * CK, CK)])

    pltpu.sync_copy(dens_v, den_s.at[pl.ds(s * 640, 640)])
    plsc.subcore_barrier()

    @pl.loop(0, CH)
    def _(ch):
        pltpu.sync_copy(src_hbm.at[s, ch], srcb.at[0])
        pltpu.sync_copy(dst_hbm.at[s, ch], dstb.at[0])
        for j in range(8):
            sl = pl.ds(j * 16, 16)
            sv = srcb[0, sl]
            dv = dstb[0, sl]
            asv = plsc.load_gather(asrc_v, [sv])
            adv = plsc.load_gather(adst_v, [dv])
            e = asv + adv
            e = jnp.where(e >= 0.0, e, 0.2 * e)
            exb[0, sl] = jnp.exp(e)
            srcb[1, sl] = sv + off
        pltpu.sync_copy(exb.at[0], den_s.at[dstb.at[0]], add=True)
        pltpu.async_copy(h_hbm.at[srcb.at[1]], rows_v, sem).wait()

        @pl.loop(0, CK // 16)
        def _(g):
            cvec = exb[0, pl.ds(g * 16, 16)]
            for k in range(16):
                cval = cvec[k]
                for f in range(8):
                    sl = pl.ds(f * 16, 16)
                    rows_v[g * 16 + k, sl] = rows_v[g * 16 + k, sl] * cval

        pltpu.sync_copy(rows_v, acc_s.at[dstb.at[0]], add=True)

    plsc.subcore_barrier()

    pltpu.sync_copy(den_s.at[pl.ds(s * 640, 640)], dens_v)

    @pl.loop(0, 640 // 16)
    def _(j):
        sl = pl.ds(j * 16, 16)
        dens_v[sl] = 1.0 / jnp.maximum(dens_v[sl], 1e-30)

    @pl.loop(0, 640 // CK)
    def _(k):
        pltpu.sync_copy(acc_s.at[pl.ds(s * 640 + k * CK, CK)], rows_v)

        @pl.loop(0, CK // 16)
        def _(g):
            cvec = dens_v[pl.ds(k * CK + g * 16, 16)]
            for t in range(16):
                cval = cvec[t]
                for f in range(8):
                    sl = pl.ds(f * 16, 16)
                    rows_v[g * 16 + t, sl] = rows_v[g * 16 + t, sl] * cval

        pltpu.sync_copy(rows_v,
                        out_hbm.at[pl.ds(c * NP + s * 640 + k * CK, CK)])


def _sc_layer(h_flat, asrc, adst, src3, dst3):
    mesh = plsc.VectorSubcoreMesh(core_axis_name="c", subcore_axis_name="s")
    fn = pl.kernel(
        _sc_body,
        out_type=jax.ShapeDtypeStruct((2 * NP, HALF), jnp.float32),
        mesh=mesh,
        compiler_params=pltpu.CompilerParams(needs_layout_passes=False),
        scratch_types=[
            pltpu.VMEM((NP,), jnp.float32),
            pltpu.VMEM((NP,), jnp.float32),
            pltpu.VMEM((2, CK), jnp.int32),
            pltpu.VMEM((1, CK), jnp.int32),
            pltpu.VMEM((1, CK), jnp.float32),
            pltpu.VMEM((CK, HALF), jnp.float32),
            pltpu.VMEM((640,), jnp.float32),
            pltpu.SemaphoreType.DMA,
            pltpu.VMEM_SHARED((NP, HALF), jnp.float32),
            pltpu.VMEM_SHARED((NP,), jnp.float32),
        ],
    )
    return fn(h_flat, asrc, adst, src3, dst3)


def kernel(x, edge_index, batch, W1, a1_src, a1_dst, b1, W2, a2_src, a2_dst,
           b2, W3, a3_src, a3_dst, b3, Wfc, bfc):
    loop = jnp.arange(N, dtype=jnp.int32)
    pad = EP - E - N
    src = jnp.concatenate([edge_index[0], loop,
                           jnp.zeros((pad,), jnp.int32)])
    dst = jnp.concatenate([edge_index[1], loop,
                           jnp.full((pad,), NP - 1, jnp.int32)])
    src3 = src.reshape(NSUB, CH, CK)
    dst3 = dst.reshape(NSUB, CH, CK)
    x_pad = jnp.pad(x, ((0, NP - N), (0, 0)))
    batch3 = jnp.pad(batch, (0, NP - N), constant_values=G).reshape(NB, 1, R)
    A1 = jnp.stack([a1_src, a1_dst], axis=1)
    A2 = jnp.stack([a2_src, a2_dst], axis=1)
    A3 = jnp.stack([a3_src, a3_dst], axis=1)

    h, alpha = _tc_layer_first(x_pad, W1, A1)
    agg = _sc_layer(h.reshape(2 * NP, HALF), alpha[:, 0], alpha[:, 1],
                    src3, dst3)
    h, alpha = _tc_layer_mid(agg, b1.reshape(1, DH), W2, A2)
    agg = _sc_layer(h.reshape(2 * NP, HALF), alpha[:, 0], alpha[:, 1],
                    src3, dst3)
    h, alpha = _tc_layer_mid(agg, b2.reshape(1, DH), W3, A3)
    agg = _sc_layer(h.reshape(2 * NP, HALF), alpha[:, 0], alpha[:, 1],
                    src3, dst3)
    return _tc_final(agg, b3.reshape(1, DH), Wfc, bfc.reshape(1, -1), batch3)


# speedup vs baseline: 16.6357x; 1.0176x over previous
import functools

import jax
import jax.numpy as jnp
from jax import lax
from jax.experimental import pallas as pl
from jax.experimental.pallas import tpu as pltpu, tpu_sc as plsc

N = 10000
NP = 10240
G = 64
DH = 256
HALF = 128
E = 320000
NSUB = 16
CK = 128
CH = 162
EP = NSUB * CH * CK
DENR = NP // 16
R = 1024
NB = NP // R


def _tc_first_body(x_ref, w_ref, a_ref, h_ref, al_ref):
    h = jnp.dot(x_ref[...], w_ref[...], preferred_element_type=jnp.float32)
    h_ref[0] = h[:, :HALF]
    h_ref[1] = h[:, HALF:]
    al_ref[...] = jnp.dot(h, a_ref[...], preferred_element_type=jnp.float32)


def _tc_mid_body(x0_ref, x1_ref, den_ref, b_ref, w_ref, a_ref, h_ref, al_ref):
    x = jnp.concatenate([x0_ref[...], x1_ref[...]], axis=1)
    x = x / jnp.maximum(den_ref[...], 1e-30)
    x = jnp.maximum(x + b_ref[...], 0.0)
    h = jnp.dot(x, w_ref[...], preferred_element_type=jnp.float32)
    h_ref[0] = h[:, :HALF]
    h_ref[1] = h[:, HALF:]
    al_ref[...] = jnp.dot(h, a_ref[...], preferred_element_type=jnp.float32)


def _tc_layer_first(x_pad, W, A):
    return pl.pallas_call(
        _tc_first_body,
        grid=(NB,),
        in_specs=[
            pl.BlockSpec((R, W.shape[0]), lambda i: (i, 0)),
            pl.BlockSpec(W.shape, lambda i: (0, 0)),
            pl.BlockSpec(A.shape, lambda i: (0, 0)),
        ],
        out_specs=[
            pl.BlockSpec((2, R, HALF), lambda i: (0, i, 0)),
            pl.BlockSpec((R, 2), lambda i: (i, 0)),
        ],
        out_shape=[
            jax.ShapeDtypeStruct((2, NP, HALF), jnp.float32),
            jax.ShapeDtypeStruct((NP, 2), jnp.float32),
        ],
    )(x_pad, W, A)


def _tc_layer_mid(agg_flat, den, b_prev, W, A):
    return pl.pallas_call(
        _tc_mid_body,
        grid=(NB,),
        in_specs=[
            pl.BlockSpec((R, HALF), lambda i: (i, 0)),
            pl.BlockSpec((R, HALF), lambda i: (i + NB, 0)),
            pl.BlockSpec((R, 1), lambda i: (i, 0)),
            pl.BlockSpec((1, DH), lambda i: (0, 0)),
            pl.BlockSpec(W.shape, lambda i: (0, 0)),
            pl.BlockSpec(A.shape, lambda i: (0, 0)),
        ],
        out_specs=[
            pl.BlockSpec((2, R, HALF), lambda i: (0, i, 0)),
            pl.BlockSpec((R, 2), lambda i: (i, 0)),
        ],
        out_shape=[
            jax.ShapeDtypeStruct((2, NP, HALF), jnp.float32),
            jax.ShapeDtypeStruct((NP, 2), jnp.float32),
        ],
    )(agg_flat, agg_flat, den, b_prev, W, A)


def _tc_final_body(x0_ref, x1_ref, den_ref, b_ref, wfc_ref, bfc_ref, bt_ref,
                   out_ref, acc_ref):
    i = pl.program_id(0)
    x = jnp.concatenate([x0_ref[...], x1_ref[...]], axis=1)
    x = x / jnp.maximum(den_ref[...], 1e-30)
    h = jnp.maximum(x + b_ref[...], 0.0)
    p = jnp.dot(h, wfc_ref[...], preferred_element_type=jnp.float32)
    bt = bt_ref[0, 0, :]
    gi = lax.broadcasted_iota(jnp.int32, (G, R), 0)
    onehot = (bt[None, :] == gi).astype(jnp.float32)
    contrib = jnp.dot(onehot, p, preferred_element_type=jnp.float32)

    @pl.when(i == 0)
    def _():
        acc_ref[...] = contrib

    @pl.when(i > 0)
    def _():
        acc_ref[...] = acc_ref[...] + contrib

    @pl.when(i == NB - 1)
    def _():
        out_ref[...] = acc_ref[...] + bfc_ref[...]


def _tc_final(agg_flat, den, b3, Wfc, bfc, batch3):
    return pl.pallas_call(
        _tc_final_body,
        grid=(NB,),
        in_specs=[
            pl.BlockSpec((R, HALF), lambda i: (i, 0)),
            pl.BlockSpec((R, HALF), lambda i: (i + NB, 0)),
            pl.BlockSpec((R, 1), lambda i: (i, 0)),
            pl.BlockSpec((1, DH), lambda i: (0, 0)),
            pl.BlockSpec(Wfc.shape, lambda i: (0, 0)),
            pl.BlockSpec((1, Wfc.shape[1]), lambda i: (0, 0)),
            pl.BlockSpec((1, 1, R), lambda i: (i, 0, 0)),
        ],
        out_specs=pl.BlockSpec((G, Wfc.shape[1]), lambda i: (0, 0)),
        out_shape=jax.ShapeDtypeStruct((G, Wfc.shape[1]), jnp.float32),
        scratch_shapes=[pltpu.VMEM((G, Wfc.shape[1]), jnp.float32)],
    )(agg_flat, agg_flat, den, b3, Wfc, bfc, batch3)


def _sc_body(h_hbm, asrc_hbm, adst_hbm, src_hbm, dst_hbm, out_hbm, den_hbm,
             asrc_v, adst_v, srcb, dstb, exb, rows_v, dens_v, sem,
             acc_s, den_s):
    c = lax.axis_index("c")
    s = lax.axis_index("s")
    zf16 = jnp.zeros((16,), jnp.float32)
    off = c * NP

    pltpu.sync_copy(asrc_hbm, asrc_v)
    pltpu.sync_copy(adst_hbm, adst_v)

    @pl.loop(0, CK)
    def _(r):
        for f in range(8):
            rows_v[r, pl.ds(f * 16, 16)] = zf16

    @pl.loop(0, 640 // 16)
    def _(j):
        dens_v[pl.ds(j * 16, 16)] = zf16

    @pl.loop(0, 640 // CK)
    def _(k):
        pltpu.sync_copy(rows_v, acc_s.at[pl.ds(s * 640 + k * CK, CK)])

    pltpu.sync_copy(dens_v, den_s.at[pl.ds(s * 640, 640)])
    plsc.subcore_barrier()

    @pl.loop(0, CH)
    def _(ch):
        pltpu.sync_copy(src_hbm.at[s, ch], srcb.at[0])
        pltpu.sync_copy(dst_hbm.at[s, ch], dstb.at[0])
        for j in range(8):
            sl = pl.ds(j * 16, 16)
            sv = srcb[0, sl]
            dv = dstb[0, sl]
            asv = plsc.load_gather(asrc_v, [sv])
            adv = plsc.load_gather(adst_v, [dv])
            e = asv + adv
            e = jnp.where(e >= 0.0, e, 0.2 * e)
            exb[0, sl] = jnp.exp(e)
            srcb[1, sl] = sv + off
        pltpu.sync_copy(exb.at[0], den_s.at[dstb.at[0]], add=True)
        pltpu.async_copy(h_hbm.at[srcb.at[1]], rows_v, sem).wait()

        @pl.loop(0, CK // 16)
        def _(g):
            cvec = exb[0, pl.ds(g * 16, 16)]
            for k in range(16):
                cval = cvec[k]
                for f in range(8):
                    sl = pl.ds(f * 16, 16)
                    rows_v[g * 16 + k, sl] = rows_v[g * 16 + k, sl] * cval

        pltpu.sync_copy(rows_v, acc_s.at[dstb.at[0]], add=True)

    plsc.subcore_barrier()

    pltpu.sync_copy(acc_s.at[pl.ds(s * 640, 640)],
                    out_hbm.at[pl.ds(c * NP + s * 640, 640)])

    @pl.when(c == 0)
    def _():
        pltpu.sync_copy(den_s.at[pl.ds(s * 640, 640)],
                        den_hbm.at[pl.ds(s * 640, 640)])


def _sc_layer(h_flat, asrc, adst, src3, dst3):
    mesh = plsc.VectorSubcoreMesh(core_axis_name="c", subcore_axis_name="s")
    fn = pl.kernel(
        _sc_body,
        out_type=[jax.ShapeDtypeStruct((2 * NP, HALF), jnp.float32),
                  jax.ShapeDtypeStruct((NP,), jnp.float32)],
        mesh=mesh,
        compiler_params=pltpu.CompilerParams(needs_layout_passes=False),
        scratch_types=[
            pltpu.VMEM((NP,), jnp.float32),
            pltpu.VMEM((NP,), jnp.float32),
            pltpu.VMEM((2, CK), jnp.int32),
            pltpu.VMEM((1, CK), jnp.int32),
            pltpu.VMEM((1, CK), jnp.float32),
            pltpu.VMEM((CK, HALF), jnp.float32),
            pltpu.VMEM((640,), jnp.float32),
            pltpu.SemaphoreType.DMA,
            pltpu.VMEM_SHARED((NP, HALF), jnp.float32),
            pltpu.VMEM_SHARED((NP,), jnp.float32),
        ],
    )
    return fn(h_flat, asrc, adst, src3, dst3)


def kernel(x, edge_index, batch, W1, a1_src, a1_dst, b1, W2, a2_src, a2_dst,
           b2, W3, a3_src, a3_dst, b3, Wfc, bfc):
    loop = jnp.arange(N, dtype=jnp.int32)
    pad = EP - E - N
    src = jnp.concatenate([edge_index[0], loop,
                           jnp.zeros((pad,), jnp.int32)])
    dst = jnp.concatenate([edge_index[1], loop,
                           jnp.full((pad,), NP - 1, jnp.int32)])
    src3 = src.reshape(NSUB, CH, CK)
    dst3 = dst.reshape(NSUB, CH, CK)
    x_pad = jnp.pad(x, ((0, NP - N), (0, 0)))
    batch3 = jnp.pad(batch, (0, NP - N), constant_values=G).reshape(NB, 1, R)
    A1 = jnp.stack([a1_src, a1_dst], axis=1)
    A2 = jnp.stack([a2_src, a2_dst], axis=1)
    A3 = jnp.stack([a3_src, a3_dst], axis=1)

    h, alpha = _tc_layer_first(x_pad, W1, A1)
    agg, den = _sc_layer(h.reshape(2 * NP, HALF), alpha[:, 0], alpha[:, 1],
                         src3, dst3)
    h, alpha = _tc_layer_mid(agg, den.reshape(NP, 1), b1.reshape(1, DH),
                             W2, A2)
    agg, den = _sc_layer(h.reshape(2 * NP, HALF), alpha[:, 0], alpha[:, 1],
                         src3, dst3)
    h, alpha = _tc_layer_mid(agg, den.reshape(NP, 1), b2.reshape(1, DH),
                             W3, A3)
    agg, den = _sc_layer(h.reshape(2 * NP, HALF), alpha[:, 0], alpha[:, 1],
                         src3, dst3)
    return _tc_final(agg, den.reshape(NP, 1), b3.reshape(1, DH), Wfc,
                     bfc.reshape(1, -1), batch3)
